# Initial kernel scaffold; baseline (speedup 1.0000x reference)
import jax
import jax.numpy as jnp
from jax.experimental import pallas as pl


def kernel(x, bin_edges_x, bin_edges_y):
    raise NotImplementedError("write your pallas kernel here")


# speedup vs baseline: 1.2577x; 1.2577x over previous
import functools

import jax
import jax.numpy as jnp
from jax.experimental import pallas as pl

_EPS = 1e-10
_BANDWIDTH = (1.0, 1.0)


def _hist_body(xt_ref, sc_ref, inv_ref, o_ref, *, n, chunk, nsteps):
    i = pl.program_id(0)
    v = xt_ref[...]
    pos = jax.lax.broadcasted_iota(jnp.int32, (1, chunk), 1) + i * chunk
    valid = pos < n
    vx = jnp.where(valid, v[0:1, :], 1e9)
    vy = jnp.where(valid, v[1:2, :], 1e9)
    scx = sc_ref[:, 0:1]
    scy = sc_ref[:, 1:2]
    ivx = inv_ref[0:1, 0:1]
    ivy = inv_ref[0:1, 1:2]
    ux = vx * ivx - scx
    uy = vy * ivy - scy
    kx = jnp.exp(ux * ux * -0.5)
    ky = jnp.exp(uy * uy * -0.5)
    p = jax.lax.dot_general(
        kx, ky, (((1,), (1,)), ((), ())), preferred_element_type=jnp.float32
    )

    @pl.when(i == 0)
    def _init():
        o_ref[...] = jnp.zeros_like(o_ref)

    o_ref[...] += p

    @pl.when(i == nsteps - 1)
    def _finalize():
        t = o_ref[...]
        o_ref[...] = t / (jnp.sum(t) + _EPS)


def kernel(x, bin_edges_x, bin_edges_y):
    n = x.shape[0]
    nb = bin_edges_x.shape[0] - 1
    cx = 0.5 * (bin_edges_x[:-1] + bin_edges_x[1:])
    cy = 0.5 * (bin_edges_y[:-1] + bin_edges_y[1:])
    sx = _BANDWIDTH[0] * (bin_edges_x[1] - bin_edges_x[0])
    sy = _BANDWIDTH[1] * (bin_edges_y[1] - bin_edges_y[0])
    sc = jnp.stack([cx / sx, cy / sy], axis=1)
    inv = jnp.stack([1.0 / sx, 1.0 / sy]).reshape(1, 2)
    xt = x[:, :2].T

    chunk = 32768
    nsteps = pl.cdiv(n, chunk)
    body = functools.partial(_hist_body, n=n, chunk=chunk, nsteps=nsteps)
    out = pl.pallas_call(
        body,
        grid=(nsteps,),
        in_specs=[
            pl.BlockSpec((2, chunk), lambda i: (0, i)),
            pl.BlockSpec((nb, 2), lambda i: (0, 0)),
            pl.BlockSpec((1, 2), lambda i: (0, 0)),
        ],
        out_specs=pl.BlockSpec((nb, nb), lambda i: (0, 0)),
        out_shape=jax.ShapeDtypeStruct((nb, nb), jnp.float32),
    )(xt, sc, inv)
    return out


# speedup vs baseline: 1.4449x; 1.1488x over previous
import functools

import jax
import jax.numpy as jnp
from jax.experimental import pallas as pl

_EPS = 1e-10
_BANDWIDTH = (1.0, 1.0)
_PAD_VAL = 1e9


def _hist_body(xt_ref, sc_ref, o_ref, *, nsteps):
    vx = xt_ref[0:1, :]
    vy = xt_ref[1:2, :]
    bx = sc_ref[:, 0:1]
    by = sc_ref[:, 1:2]
    kx = jnp.exp2((bx - vx) * (vx - bx)).astype(jnp.bfloat16)
    ky = jnp.exp2((by - vy) * (vy - by)).astype(jnp.bfloat16)
    p = jax.lax.dot_general(
        kx, ky, (((1,), (1,)), ((), ())), preferred_element_type=jnp.float32
    )

    i = pl.program_id(0)

    @pl.when(i == 0)
    def _init():
        o_ref[...] = jnp.zeros_like(o_ref)

    o_ref[...] += p

    @pl.when(i == nsteps - 1)
    def _finalize():
        t = o_ref[...]
        o_ref[...] = t / (jnp.sum(t) + _EPS)


def kernel(x, bin_edges_x, bin_edges_y):
    n = x.shape[0]
    nb = bin_edges_x.shape[0] - 1
    cx = 0.5 * (bin_edges_x[:-1] + bin_edges_x[1:])
    cy = 0.5 * (bin_edges_y[:-1] + bin_edges_y[1:])
    sx = _BANDWIDTH[0] * (bin_edges_x[1] - bin_edges_x[0])
    sy = _BANDWIDTH[1] * (bin_edges_y[1] - bin_edges_y[0])
    root = jnp.sqrt(jnp.float32(0.5 / jnp.log(2.0)))
    ax = root / sx
    ay = root / sy
    sc = jnp.stack([cx * ax, cy * ay], axis=1)

    chunk = 65536
    nsteps = pl.cdiv(n, chunk)
    total = nsteps * chunk
    xt = jnp.pad(
        (x[:, :2] * jnp.stack([ax, ay])).T,
        ((0, 0), (0, total - n)),
        constant_values=_PAD_VAL,
    )

    body = functools.partial(_hist_body, nsteps=nsteps)
    out = pl.pallas_call(
        body,
        grid=(nsteps,),
        in_specs=[
            pl.BlockSpec((2, chunk), lambda i: (0, i)),
            pl.BlockSpec((nb, 2), lambda i: (0, 0)),
        ],
        out_specs=pl.BlockSpec((nb, nb), lambda i: (0, 0)),
        out_shape=jax.ShapeDtypeStruct((nb, nb), jnp.float32),
    )(xt, sc)
    return out
